# baseline (speedup 1.0000x reference)
import functools

import jax
import jax.numpy as jnp
from jax import lax
from jax.experimental import pallas as pl
from jax.experimental.pallas import tpu as pltpu
from jax.experimental.pallas import tpu_sc as plsc

BATCH = 16384
A = 26
V = 100000
D = 16
L = 16

NC = 2
NS = 16
NW = NC * NS
ROWS = BATCH * A
SB = 64
CROWS = SB * A
NCHUNK = (BATCH // NW) // SB
NG = CROWS // 128
NVREG = CROWS // L
PERIOD = 13


@functools.partial(
    pl.kernel,
    out_type=jax.ShapeDtypeStruct((ROWS, D), jnp.float32),
    mesh=plsc.VectorSubcoreMesh(core_axis_name="c", subcore_axis_name="s"),
    compiler_params=pltpu.CompilerParams(use_tc_tiling_on_sc=False),
    scratch_types=[
        pltpu.VMEM((CROWS,), jnp.int32),
        pltpu.VMEM((NG, 128), jnp.int32),
        pltpu.VMEM((CROWS, D), jnp.float32),
        pltpu.VMEM((A, D), jnp.float32),
        pltpu.VMEM((A, D), jnp.float32),
        pltpu.SemaphoreType.DMA,
    ],
)
def _sc_tokenize(xf_hbm, tab_hbm, scale_hbm, bias_hbm, out_hbm,
                 xv, idxv, rows, scale_v, bias_v, sem):
    wid = lax.axis_index("s") * NC + lax.axis_index("c")

    pltpu.sync_copy(scale_hbm, scale_v)
    pltpu.sync_copy(bias_hbm, bias_v)

    iota = lax.iota(jnp.int32, L)
    offs = [((iota + (m * L) % A) % A) * V for m in range(PERIOD)]

    def chunk_body(t, carry):
        base = (wid * NCHUNK + t) * CROWS
        pltpu.sync_copy(xf_hbm.at[pl.ds(base, CROWS)], xv)
        for g in range(NVREG):
            r, c = g // 8, (g % 8) * L
            idxv[r, pl.ds(c, L)] = xv[pl.ds(g * L, L)] + offs[g % PERIOD]
        cps = [
            pltpu.async_copy(tab_hbm.at[idxv.at[j]],
                             rows.at[pl.ds(j * 128, 128)], sem)
            for j in range(NG)
        ]
        for cp in cps:
            cp.wait()

        def row_group(q, c2):
            rbase = q * A
            for i in range(A):
                r = rbase + i
                rows[r] = rows[r] * scale_v[i] + bias_v[i]
            return c2

        lax.fori_loop(0, SB, row_group, 0)
        pltpu.sync_copy(rows, out_hbm.at[pl.ds(base, CROWS)])
        return carry

    lax.fori_loop(0, NCHUNK, chunk_body, 0)


def kernel(x, attr_emb, option_tables, prior):
    xf = x.reshape(-1).astype(jnp.int32)
    tab = option_tables.reshape(A * V, D)
    scale = jnp.broadcast_to(prior, (A, D)).astype(jnp.float32)
    bias = (attr_emb * prior).astype(jnp.float32)
    out = _sc_tokenize(xf, tab, scale, bias)
    return out.reshape(BATCH, A, D)


# speedup vs baseline: 7.1086x; 7.1086x over previous
import functools

import jax
import jax.numpy as jnp
from jax import lax
from jax.experimental import pallas as pl
from jax.experimental.pallas import tpu as pltpu
from jax.experimental.pallas import tpu_sc as plsc

BATCH = 16384
A = 26
V = 100000
D = 16
L = 16

NC = 2
NS = 16
NW = NC * NS
NTASK = A * D
TPW = NTASK // NW
BCH = 4096
NBCH = BATCH // BCH


@functools.partial(
    pl.kernel,
    out_type=jax.ShapeDtypeStruct((A, D, BATCH), jnp.float32),
    mesh=plsc.VectorSubcoreMesh(core_axis_name="c", subcore_axis_name="s"),
    compiler_params=pltpu.CompilerParams(
        use_tc_tiling_on_sc=True, needs_layout_passes=False),
    scratch_types=[
        pltpu.VMEM((V,), jnp.float32),
        pltpu.VMEM((BCH,), jnp.int32),
        pltpu.VMEM((BCH,), jnp.float32),
        pltpu.VMEM((NTASK,), jnp.float32),
        pltpu.VMEM((NTASK,), jnp.float32),
    ],
)
def _sc_tokenize(xt_hbm, tab_hbm, scale_hbm, bias_hbm, out_hbm,
                 rowbuf, xv, outv, scale_v, bias_v):
    wid = lax.axis_index("s") * NC + lax.axis_index("c")

    pltpu.sync_copy(scale_hbm, scale_v)
    pltpu.sync_copy(bias_hbm, bias_v)

    def task_body(t, carry):
        task = wid * TPW + t
        i = task // D
        d = task % D
        pltpu.sync_copy(tab_hbm.at[i, d], rowbuf)
        tsplat = jnp.full((L,), task, dtype=jnp.int32)
        s = plsc.load_gather(scale_v, [tsplat])
        b = plsc.load_gather(bias_v, [tsplat])

        def chunk_body(c, c2):
            pltpu.sync_copy(xt_hbm.at[i, pl.ds(c * BCH, BCH)], xv)

            def grp(g, c3):
                idx = xv[pl.ds(g * L, L)]
                vals = plsc.load_gather(rowbuf, [idx])
                outv[pl.ds(g * L, L)] = vals * s + b
                return c3

            lax.fori_loop(0, BCH // L, grp, 0)
            pltpu.sync_copy(outv, out_hbm.at[i, d, pl.ds(c * BCH, BCH)])
            return c2

        lax.fori_loop(0, NBCH, chunk_body, 0)
        return carry

    lax.fori_loop(0, TPW, task_body, 0)


def kernel(x, attr_emb, option_tables, prior):
    xt = x.T.astype(jnp.int32)
    tab = option_tables.transpose(0, 2, 1)
    scale = jnp.broadcast_to(prior, (A, D)).astype(jnp.float32).reshape(-1)
    bias = (attr_emb * prior).astype(jnp.float32).reshape(-1)
    out3 = _sc_tokenize(xt, tab, scale, bias)
    return out3.transpose(2, 0, 1)


# speedup vs baseline: 8.5580x; 1.2039x over previous
import functools

import jax
import jax.numpy as jnp
from jax import lax
from jax.experimental import pallas as pl
from jax.experimental.pallas import tpu as pltpu
from jax.experimental.pallas import tpu_sc as plsc

BATCH = 16384
A = 26
V = 100000
D = 16
L = 16

NC = 2
NS = 16
NW = NC * NS
NTASK = A * D
TPW = NTASK // NW
BCH = 4096
NBCH = BATCH // BCH
UNROLL = 8


@functools.partial(
    pl.kernel,
    out_type=jax.ShapeDtypeStruct((A, D, BATCH), jnp.float32),
    mesh=plsc.VectorSubcoreMesh(core_axis_name="c", subcore_axis_name="s"),
    compiler_params=pltpu.CompilerParams(
        use_tc_tiling_on_sc=True, needs_layout_passes=False),
    scratch_types=[
        pltpu.VMEM((V,), jnp.float32),
        pltpu.VMEM((BATCH,), jnp.int32),
        pltpu.VMEM((BCH,), jnp.float32),
        pltpu.VMEM((BCH,), jnp.float32),
        pltpu.VMEM((NTASK,), jnp.float32),
        pltpu.VMEM((NTASK,), jnp.float32),
        pltpu.SemaphoreType.DMA,
        pltpu.SemaphoreType.DMA,
        pltpu.SemaphoreType.DMA,
    ],
)
def _sc_tokenize(xt_hbm, tab_hbm, scale_hbm, bias_hbm, out_hbm,
                 rowbuf, xv, outv0, outv1, scale_v, bias_v,
                 sem_row, sem_x, sem_out):
    wid = lax.axis_index("s") * NC + lax.axis_index("c")

    pltpu.sync_copy(scale_hbm, scale_v)
    pltpu.sync_copy(bias_hbm, bias_v)

    def task_body(t, carry):
        task = wid * TPW + t
        i = task // D
        d = task % D
        cp_row = pltpu.async_copy(tab_hbm.at[i, d], rowbuf, sem_row)
        cp_x = pltpu.async_copy(xt_hbm.at[i], xv, sem_x)
        tsplat = jnp.full((L,), task, dtype=jnp.int32)
        s = plsc.load_gather(scale_v, [tsplat])
        b = plsc.load_gather(bias_v, [tsplat])
        cp_row.wait()
        cp_x.wait()

        outcps = []
        bufs = (outv0, outv1)
        for c in range(NBCH):
            buf = bufs[c % 2]
            if c >= 2:
                outcps[c - 2].wait()

            def grp(g, c3, c=c, buf=buf):
                for u in range(UNROLL):
                    o = g * (L * UNROLL) + u * L
                    idx = xv[pl.ds(c * BCH + o, L)]
                    vals = plsc.load_gather(rowbuf, [idx])
                    buf[pl.ds(o, L)] = vals * s + b
                return c3

            lax.fori_loop(0, BCH // (L * UNROLL), grp, 0)
            outcps.append(pltpu.async_copy(
                buf, out_hbm.at[i, d, pl.ds(c * BCH, BCH)], sem_out))
        outcps[NBCH - 2].wait()
        outcps[NBCH - 1].wait()
        return carry

    lax.fori_loop(0, TPW, task_body, 0)


def kernel(x, attr_emb, option_tables, prior):
    xt = x.T.astype(jnp.int32)
    tab = option_tables.transpose(0, 2, 1)
    scale = jnp.broadcast_to(prior, (A, D)).astype(jnp.float32).reshape(-1)
    bias = (attr_emb * prior).astype(jnp.float32).reshape(-1)
    out3 = _sc_tokenize(xt, tab, scale, bias)
    return out3.transpose(2, 0, 1)


# speedup vs baseline: 13.4047x; 1.5663x over previous
import functools

import jax
import jax.numpy as jnp
from jax import lax
from jax.experimental import pallas as pl
from jax.experimental.pallas import tpu as pltpu
from jax.experimental.pallas import tpu_sc as plsc

BATCH = 16384
A = 26
V = 100000
D = 16
L = 16

NC = 2
NS = 16
NW = NC * NS
NTASK = A * D
TPW = NTASK // NW
BCH = 4096
NBCH = BATCH // BCH
UNROLL = 8


@functools.partial(
    pl.kernel,
    out_type=jax.ShapeDtypeStruct((A, D, BATCH), jnp.float32),
    mesh=plsc.VectorSubcoreMesh(core_axis_name="c", subcore_axis_name="s"),
    compiler_params=pltpu.CompilerParams(
        use_tc_tiling_on_sc=True, needs_layout_passes=False),
    scratch_types=[
        pltpu.VMEM((V,), jnp.float32),
        pltpu.VMEM((BATCH,), jnp.int32),
        pltpu.VMEM((BCH,), jnp.float32),
        pltpu.VMEM((BCH,), jnp.float32),
        pltpu.VMEM((NTASK,), jnp.float32),
        pltpu.VMEM((NTASK,), jnp.float32),
        pltpu.SemaphoreType.DMA,
        pltpu.SemaphoreType.DMA,
        pltpu.SemaphoreType.DMA,
    ],
)
def _sc_tokenize(xt_hbm, tab_hbm, scale_hbm, bias_hbm, out_hbm,
                 rowbuf, xv, outv0, outv1, scale_v, bias_v,
                 sem_row, sem_x, sem_out):
    wid = lax.axis_index("s") * NC + lax.axis_index("c")

    pltpu.sync_copy(scale_hbm, scale_v)
    pltpu.sync_copy(bias_hbm, bias_v)

    def task_body(t, carry):
        task = wid * TPW + t
        i = task // D
        d = task % D
        cp_row = pltpu.async_copy(tab_hbm.at[i, d], rowbuf, sem_row)
        cp_x = pltpu.async_copy(xt_hbm.at[i], xv, sem_x)
        tsplat = jnp.full((L,), task, dtype=jnp.int32)
        s = plsc.load_gather(scale_v, [tsplat])
        b = plsc.load_gather(bias_v, [tsplat])
        cp_row.wait()
        cp_x.wait()

        outcps = []
        bufs = (outv0, outv1)
        for c in range(NBCH):
            buf = bufs[c % 2]
            if c >= 2:
                outcps[c - 2].wait()

            @plsc.parallel_loop(0, BCH // L, unroll=UNROLL)
            def grp(g, c=c, buf=buf):
                o = g * L
                idx = xv[pl.ds(c * BCH + o, L)]
                vals = plsc.load_gather(rowbuf, [idx])
                buf[pl.ds(o, L)] = vals * s + b
            outcps.append(pltpu.async_copy(
                buf, out_hbm.at[i, d, pl.ds(c * BCH, BCH)], sem_out))
        outcps[NBCH - 2].wait()
        outcps[NBCH - 1].wait()
        return carry

    lax.fori_loop(0, TPW, task_body, 0)


def kernel(x, attr_emb, option_tables, prior):
    xt = x.T.astype(jnp.int32)
    tab = option_tables.transpose(0, 2, 1)
    scale = jnp.broadcast_to(prior, (A, D)).astype(jnp.float32).reshape(-1)
    bias = (attr_emb * prior).astype(jnp.float32).reshape(-1)
    out3 = _sc_tokenize(xt, tab, scale, bias)
    return out3.transpose(2, 0, 1)
